# baseline (speedup 1.0000x reference)
import functools
import jax
import jax.numpy as jnp
from jax import lax
from jax.experimental import pallas as pl
from jax.experimental.pallas import tpu as pltpu
from jax.experimental.pallas import tpu_sc as plsc

_BINS = 16
_B_SC = 4
_NW = 32
_LN2 = 0.6931471805599453


def _dfl_body(x_ref, d_ref, m_ref, tot_ref, npos_ref):
    b = pl.program_id(0)
    pm = m_ref[0].astype(jnp.float32)
    partial = jnp.zeros((), jnp.float32)
    for s in range(4):
        d = jnp.clip(d_ref[0, s], 0.0, float(_BINS - 1))
        base = s * _BINS
        mx = x_ref[0, base]
        for k in range(1, _BINS):
            mx = jnp.maximum(mx, x_ref[0, base + k])
        xp = x_ref[0, base]
        ssum = jnp.exp(xp - mx)
        acc = xp
        for k in range(1, _BINS):
            xk = x_ref[0, base + k]
            ssum += jnp.exp(xk - mx)
            acc += jnp.clip(d - float(k - 1), 0.0, 1.0) * (xk - xp)
            xp = xk
        lse = jnp.log(ssum) + mx
        partial += jnp.sum((lse - acc) * pm)

    @pl.when(b == 0)
    def _init():
        tot_ref[0, 0] = 0.0
        npos_ref[0, 0] = 0.0

    tot_ref[0, 0] += partial
    npos_ref[0, 0] += jnp.sum(pm)


def _tc_part(x, dist_t, pm, H, W):
    B, C = x.shape[0], x.shape[1]
    tot, npos = pl.pallas_call(
        _dfl_body,
        grid=(B,),
        in_specs=[
            pl.BlockSpec((1, C, H, W), lambda b: (b, 0, 0, 0)),
            pl.BlockSpec((1, 4, H, W), lambda b: (b, 0, 0, 0)),
            pl.BlockSpec((1, H, W), lambda b: (b, 0, 0)),
        ],
        out_specs=[
            pl.BlockSpec(memory_space=pltpu.SMEM),
            pl.BlockSpec(memory_space=pltpu.SMEM),
        ],
        out_shape=[
            jax.ShapeDtypeStruct((1, 1), jnp.float32),
            jax.ShapeDtypeStruct((1, 1), jnp.float32),
        ],
    )(x, dist_t, pm)
    return tot[0, 0], npos[0, 0]


def _log_pos(x):
    bits = lax.bitcast_convert_type(x, jnp.int32)
    e = lax.shift_right_logical(bits, 23) - 127
    m = lax.bitcast_convert_type(
        jnp.bitwise_or(jnp.bitwise_and(bits, 0x007FFFFF), 0x3F800000),
        jnp.float32)
    z = (m - 1.0) / (m + 1.0)
    z2 = z * z
    p = z * (2.0 + z2 * (0.66666666 + z2 * (0.4 + z2 * (0.28571429
        + z2 * 0.22222222))))
    return p + _LN2 * e.astype(jnp.float32)


def _sc_loss_call(x_hbm_shape, chunk):
    mesh = plsc.VectorSubcoreMesh(core_axis_name="c", subcore_axis_name="s")
    b_sc = x_hbm_shape[0]

    @functools.partial(
        pl.kernel,
        mesh=mesh,
        compiler_params=pltpu.CompilerParams(use_tc_tiling_on_sc=False),
        out_type=[
            jax.ShapeDtypeStruct((_NW, 16), jnp.float32),
            jax.ShapeDtypeStruct((_NW, 16), jnp.float32),
        ],
        scratch_types=[
            pltpu.VMEM((64 * chunk,), jnp.float32),
            pltpu.VMEM((4 * chunk,), jnp.float32),
            pltpu.VMEM((chunk,), jnp.float32),
            pltpu.VMEM((16,), jnp.float32),
            pltpu.VMEM((16,), jnp.float32),
            pltpu.SemaphoreType.DMA,
        ],
    )
    def sc_kernel(x_hbm, d_hbm, pm_hbm, out_loss, out_npos, x_v, d_v, m_v,
                  res_v, np_v, sem):
        wid = lax.axis_index("s") * 2 + lax.axis_index("c")
        base = wid * chunk
        hw = chunk * _NW
        lanes = lax.broadcasted_iota(jnp.int32, (16,), 0)

        loss_acc = jnp.zeros((16,), jnp.float32)
        np_acc = jnp.zeros((16,), jnp.float32)
        for b in range(b_sc):
            copies = [
                pltpu.async_copy(
                    x_hbm.at[b, pl.ds(c * hw + base, chunk)],
                    x_v.at[pl.ds(c * chunk, chunk)], sem)
                for c in range(64)
            ]
            copies.append(pltpu.async_copy(
                d_hbm.at[b, pl.ds(0 * hw + base, chunk)],
                d_v.at[pl.ds(0, chunk)], sem))
            for s in range(1, 4):
                copies.append(pltpu.async_copy(
                    d_hbm.at[b, pl.ds(s * hw + base, chunk)],
                    d_v.at[pl.ds(s * chunk, chunk)], sem))
            copies.append(pltpu.async_copy(
                pm_hbm.at[b, pl.ds(base, chunk)], m_v, sem))
            for cp in copies:
                cp.wait()

            def chunk_body(i, carry):
                loss_acc, np_acc = carry
                col = i * 16
                pm_vec = m_v[pl.ds(col, 16)]
                np_acc = np_acc + pm_vec
                for s in range(4):
                    row = s * _BINS
                    mx = x_v[pl.ds(row * chunk + col, 16)]
                    for k in range(1, _BINS):
                        mx = jnp.maximum(
                            mx, x_v[pl.ds((row + k) * chunk + col, 16)])
                    d = jnp.clip(d_v[pl.ds(s * chunk + col, 16)],
                                 0.0, float(_BINS - 1))
                    xp = x_v[pl.ds(row * chunk + col, 16)]
                    ssum = jnp.exp(xp - mx)
                    acc = xp
                    for k in range(1, _BINS):
                        xk = x_v[pl.ds((row + k) * chunk + col, 16)]
                        ssum = ssum + jnp.exp(xk - mx)
                        acc = acc + jnp.clip(d - float(k - 1), 0.0, 1.0) * (xk - xp)
                        xp = xk
                    lse = _log_pos(ssum) + mx
                    loss_acc = loss_acc + (lse - acc) * pm_vec
                return loss_acc, np_acc

            loss_acc, np_acc = lax.fori_loop(
                0, chunk // 16, chunk_body, (loss_acc, np_acc))

        res_v[...] = loss_acc
        np_v[...] = np_acc
        pltpu.sync_copy(res_v, out_loss.at[wid])
        pltpu.sync_copy(np_v, out_npos.at[wid])

    return sc_kernel


@jax.jit
def kernel(reg_logits, dist_targets, pos_mask):
    B, C, H, W = reg_logits.shape
    HW = H * W
    dist_t = jnp.transpose(dist_targets, (0, 2, 1)).reshape(B, 4, H, W)
    pm = pos_mask.reshape(B, H, W)

    chunk = HW // _NW
    x_sc = reg_logits[:_B_SC].reshape(_B_SC, C * HW)
    d_sc = dist_t[:_B_SC].reshape(_B_SC, 4 * HW)
    pm_sc = pos_mask[:_B_SC].astype(jnp.float32)
    sc_loss, sc_np = _sc_loss_call(x_sc.shape, chunk)(x_sc, d_sc, pm_sc)

    tc_tot, tc_np = _tc_part(reg_logits[_B_SC:], dist_t[_B_SC:], pm[_B_SC:],
                             H, W)

    total = tc_tot + jnp.sum(sc_loss)
    n_pos = tc_np + jnp.sum(sc_np)
    return jnp.where(n_pos > 0, total / jnp.maximum(n_pos * 4.0, 1.0), 0.0)


# speedup vs baseline: 2.3662x; 2.3662x over previous
import jax
import jax.numpy as jnp
from jax.experimental import pallas as pl
from jax.experimental.pallas import tpu as pltpu

_BINS = 16


def _dfl_body(x_ref, d_ref, m_ref, tot_ref, npos_ref):
    b = pl.program_id(0)
    pm = m_ref[0].astype(jnp.float32)
    partial = jnp.zeros((), jnp.float32)
    for s in range(4):
        d = jnp.clip(d_ref[0, s], 0.0, float(_BINS - 1))
        base = s * _BINS
        mx = x_ref[0, base]
        for k in range(1, _BINS):
            mx = jnp.maximum(mx, x_ref[0, base + k])
        xp = x_ref[0, base]
        ssum = jnp.exp(xp - mx)
        acc = xp
        for k in range(1, _BINS):
            xk = x_ref[0, base + k]
            ssum += jnp.exp(xk - mx)
            acc += jnp.clip(d - float(k - 1), 0.0, 1.0) * (xk - xp)
            xp = xk
        lse = jnp.log(ssum) + mx
        partial += jnp.sum((lse - acc) * pm)

    @pl.when(b == 0)
    def _init():
        tot_ref[0, 0] = 0.0
        npos_ref[0, 0] = 0.0

    tot_ref[0, 0] += partial
    npos_ref[0, 0] += jnp.sum(pm)


@jax.jit
def kernel(reg_logits, dist_targets, pos_mask):
    B, C, H, W = reg_logits.shape
    dist_t = jnp.transpose(dist_targets, (0, 2, 1)).reshape(B, 4, H, W)
    pm = pos_mask.reshape(B, H, W)

    tot, npos = pl.pallas_call(
        _dfl_body,
        grid=(B,),
        in_specs=[
            pl.BlockSpec((1, C, H, W), lambda b: (b, 0, 0, 0)),
            pl.BlockSpec((1, 4, H, W), lambda b: (b, 0, 0, 0)),
            pl.BlockSpec((1, H, W), lambda b: (b, 0, 0)),
        ],
        out_specs=[
            pl.BlockSpec(memory_space=pltpu.SMEM),
            pl.BlockSpec(memory_space=pltpu.SMEM),
        ],
        out_shape=[
            jax.ShapeDtypeStruct((1, 1), jnp.float32),
            jax.ShapeDtypeStruct((1, 1), jnp.float32),
        ],
    )(reg_logits, dist_t, pm)

    total = tot[0, 0]
    n_pos = npos[0, 0]
    return jnp.where(n_pos > 0, total / jnp.maximum(n_pos * 4.0, 1.0), 0.0)
